# Initial kernel scaffold; baseline (speedup 1.0000x reference)
import jax
import jax.numpy as jnp
from jax.experimental import pallas as pl


def kernel(x, edge_index, batch, edge_attr, fields, Wl1, bl1, Wr1, Wl2, bl2, Wr2, Wl3, bl3, Wr3, p1, p2, p3, W1, b1, W2, b2, W3, b3):
    raise NotImplementedError("write your pallas kernel here")


# speedup vs baseline: 9.6249x; 9.6249x over previous
import functools
import math

import numpy as np
import jax
import jax.numpy as jnp
from jax import lax
from jax.experimental import pallas as pl
from jax.experimental.pallas import tpu as pltpu
from jax.experimental.pallas import tpu_sc as plsc

_N = 50000
_E = 800000
_F = 64
_HF = 32

_NSC = 2
_NTILE = 16
_CH = 128

_N_PAD = 50176
_RPT = _N_PAD // _NTILE
_ZR = 392
_E_PAD = 800768
_EPT = _E_PAD // _NTILE
_NCHUNK = _EPT // _CH

_BLK = 512
_NBLK = _N_PAD // _BLK
_SROW = _N_PAD // 128

_MIN_I32 = np.int32(-2147483648)
_F32 = jnp.float32

_TRIL512 = np.tril(np.ones((_BLK, _BLK), np.float32), -1)


def _sc_segment_sum(hzA, hzB, alive, srcp, dstp):
  mesh = plsc.VectorSubcoreMesh(
      core_axis_name="c", subcore_axis_name="s",
      num_cores=_NSC, num_subcores=_NTILE)
  out_type = (
      jax.ShapeDtypeStruct((_N_PAD, _HF), _F32),
      jax.ShapeDtypeStruct((_N_PAD, _HF), _F32),
      jax.ShapeDtypeStruct((_N_PAD,), _F32),
      jax.ShapeDtypeStruct((_N_PAD,), _F32),
  )
  scratch = [
      pltpu.VMEM((_CH,), jnp.int32),
      pltpu.VMEM((_CH,), jnp.int32),
      pltpu.VMEM((_CH, _HF), _F32),
      pltpu.VMEM((_CH,), _F32),
      pltpu.VMEM((_ZR, _HF), _F32),
      pltpu.VMEM((_RPT,), _F32),
      pltpu.VMEM_SHARED((_N_PAD, _HF), _F32),
      pltpu.VMEM_SHARED((_N_PAD,), _F32),
      pltpu.SemaphoreType.DMA,
  ]

  @functools.partial(
      pl.kernel, out_type=out_type, mesh=mesh, scratch_types=scratch,
      compiler_params=pltpu.CompilerParams(use_tc_tiling_on_sc=False))
  def k(hzA_h, hzB_h, alive_h, src_h, dst_h,
        ssumA_h, ssumB_h, degA_h, degB_h,
        src_buf, dst_buf, val_buf, a_buf, zbuf, zbufd,
        acc, deg_acc, sem):
    c = lax.axis_index("c")
    s = lax.axis_index("s")
    zv = jnp.zeros((16,), _F32)

    def fill_z(i, carry):
      zbuf[i // 2, pl.ds((i % 2) * 16, 16)] = zv
      return carry
    lax.fori_loop(0, _ZR * 2, fill_z, 0)

    def fill_zd(i, carry):
      zbufd[pl.ds(i * 16, 16)] = zv
      return carry
    lax.fori_loop(0, _RPT // 16, fill_zd, 0)

    row0 = s * _RPT
    for b in range(_RPT // _ZR):
      pltpu.sync_copy(zbuf, acc.at[pl.ds(row0 + b * _ZR, _ZR)])
    pltpu.sync_copy(zbufd, deg_acc.at[pl.ds(row0, _RPT)])
    plsc.subcore_barrier()

    ebase = s * _EPT

    def body(j, carry):
      base = ebase + j * _CH
      pltpu.sync_copy(src_h.at[pl.ds(base, _CH)], src_buf)
      pltpu.sync_copy(dst_h.at[pl.ds(base, _CH)], dst_buf)

      @pl.when(c == 0)
      def _():
        pltpu.async_copy(hzA_h.at[src_buf], val_buf, sem).wait()

      @pl.when(c == 1)
      def _():
        pltpu.async_copy(hzB_h.at[src_buf], val_buf, sem).wait()

      pltpu.sync_copy(val_buf, acc.at[dst_buf], add=True)

      @pl.when(j % 2 == c)
      def _():
        pltpu.async_copy(alive_h.at[src_buf], a_buf, sem).wait()
        pltpu.sync_copy(a_buf, deg_acc.at[dst_buf], add=True)
      return carry
    lax.fori_loop(0, _NCHUNK, body, 0)
    plsc.subcore_barrier()

    @pl.when(c == 0)
    def _():
      pltpu.sync_copy(acc.at[pl.ds(row0, _RPT)],
                      ssumA_h.at[pl.ds(row0, _RPT)])
      pltpu.sync_copy(deg_acc.at[pl.ds(row0, _RPT)],
                      degA_h.at[pl.ds(row0, _RPT)])

    @pl.when(c == 1)
    def _():
      pltpu.sync_copy(acc.at[pl.ds(row0, _RPT)],
                      ssumB_h.at[pl.ds(row0, _RPT)])
      pltpu.sync_copy(deg_acc.at[pl.ds(row0, _RPT)],
                      degB_h.at[pl.ds(row0, _RPT)])

  return k(hzA, hzB, alive, srcp, dstp)


def _gelu(t):
  return 0.5 * t * (1.0 + lax.erf(t * np.float32(1.0 / math.sqrt(2.0))))


def _tc_sage(ssumA, ssumB, degA, degB, hzA, hzB, alive, Wl, bl, Wr, p):
  def body(sA, sB, dA, dB, hA, hB, al, wl, blv, wr, pv, hp_ref, sc_ref):
    d = jnp.maximum(dA[...] + dB[...], 1.0)
    inv = 1.0 / d
    mean = jnp.concatenate([sA[...], sB[...]], axis=1) * inv
    hz = jnp.concatenate([hA[...], hB[...]], axis=1)
    pre = (jnp.dot(mean, wl[...], preferred_element_type=_F32) + blv[...]
           + jnp.dot(hz, wr[...], preferred_element_type=_F32))
    hp = _gelu(pre)
    pn = pv[...]
    pn = pn * lax.rsqrt(jnp.sum(pn * pn))
    sco = jnp.tanh(jnp.sum(hp * pn, axis=1, keepdims=True))
    hp_ref[...] = hp
    sc_ref[...] = jnp.where(al[...] > 0.0, sco, -2.0)

  nblock = lambda cols: pl.BlockSpec((_BLK, cols), lambda i: (i, 0))
  wblock = lambda r, c: pl.BlockSpec((r, c), lambda i: (0, 0))
  return pl.pallas_call(
      body,
      grid=(_NBLK,),
      in_specs=[nblock(_HF), nblock(_HF), nblock(1), nblock(1),
                nblock(_HF), nblock(_HF), nblock(1),
                wblock(_F, _F), wblock(1, _F), wblock(_F, _F), wblock(1, _F)],
      out_specs=[nblock(_F), nblock(1)],
      out_shape=[jax.ShapeDtypeStruct((_N_PAD, _F), _F32),
                 jax.ShapeDtypeStruct((_N_PAD, 1), _F32)],
  )(ssumA, ssumB, degA, degB, hzA, hzB, alive, Wl, bl, Wr, p)


def _monotone_i32(bits):
  return jnp.where(bits >= 0, bits, bits ^ np.int32(0x7FFFFFFF))


def _tc_thresh(scg, kk):
  def body(sc_ref, thr_ref, tn_ref):
    mi = _monotone_i32(lax.bitcast_convert_type(sc_ref[...], jnp.int32))

    def step(i, P):
      bv = jnp.left_shift(np.int32(1), 31 - i)
      cand = P | bv
      cand_s = cand ^ _MIN_I32
      cnt = jnp.sum((mi >= cand_s).astype(jnp.int32))
      return jnp.where(cnt >= kk, cand, P)

    P = lax.fori_loop(0, 32, step, jnp.int32(0))
    T = P ^ _MIN_I32
    cg = jnp.sum((mi > T).astype(jnp.int32))
    thr_ref[...] = jnp.broadcast_to(T, (1, 1))
    tn_ref[...] = jnp.broadcast_to(kk - cg, (1, 1))

  return pl.pallas_call(
      body,
      out_shape=[jax.ShapeDtypeStruct((1, 1), jnp.int32),
                 jax.ShapeDtypeStruct((1, 1), jnp.int32)],
  )(scg)


def _tc_select(hp, sc, thr, tn, tril, kk):
  def body(hp_ref, sc_ref, thr_ref, tn_ref, tril_ref,
           hzA_ref, hzB_ref, al_ref, x_ref, cnt_ref):
    i = pl.program_id(0)

    @pl.when(i == 0)
    def _():
      cnt_ref[0] = np.int32(0)
      x_ref[...] = jnp.concatenate(
          [jnp.full((1, _F), -1e30, _F32), jnp.zeros((1, _F), _F32)], axis=1)

    scb = sc_ref[...]
    mi = _monotone_i32(lax.bitcast_convert_type(scb, jnp.int32))
    T = thr_ref[...]
    tnf = tn_ref[...].astype(_F32)
    eq = mi == T
    eqf = eq.astype(_F32)
    base = cnt_ref[0].astype(_F32)
    pref = jnp.dot(tril_ref[...], eqf, preferred_element_type=_F32) + base
    sel = (mi > T) | (eq & (pref < tnf))
    cnt_ref[0] = cnt_ref[0] + jnp.sum(eqf).astype(jnp.int32)

    selF = sel.astype(_F32)
    hz = hp_ref[...] * (selF * scb)
    hzA_ref[...] = hz[:, :_HF]
    hzB_ref[...] = hz[:, _HF:]
    al_ref[...] = selF
    pmax = jnp.max(jnp.where(sel, hz, -1e30), axis=0, keepdims=True)
    psum = jnp.sum(hz, axis=0, keepdims=True)
    old = x_ref[...]
    x_ref[...] = jnp.concatenate(
        [jnp.maximum(old[:, :_F], pmax), old[:, _F:] + psum], axis=1)

    @pl.when(i == _NBLK - 1)
    def _():
      fin = x_ref[...]
      x_ref[...] = jnp.concatenate(
          [fin[:, :_F], fin[:, _F:] * np.float32(1.0 / kk)], axis=1)

  nblock = lambda cols: pl.BlockSpec((_BLK, cols), lambda i: (i, 0))
  full = lambda r, c: pl.BlockSpec((r, c), lambda i: (0, 0))
  return pl.pallas_call(
      body,
      grid=(_NBLK,),
      in_specs=[nblock(_F), nblock(1), full(1, 1), full(1, 1),
                full(_BLK, _BLK)],
      out_specs=[nblock(_HF), nblock(_HF), nblock(1), full(1, 2 * _F)],
      out_shape=[jax.ShapeDtypeStruct((_N_PAD, _HF), _F32),
                 jax.ShapeDtypeStruct((_N_PAD, _HF), _F32),
                 jax.ShapeDtypeStruct((_N_PAD, 1), _F32),
                 jax.ShapeDtypeStruct((1, 2 * _F), _F32)],
      scratch_shapes=[pltpu.SMEM((1,), jnp.int32)],
  )(hp, sc, thr, tn, tril)


def _tc_mlp(z, W1, b1, W2, b2, W3, b3):
  def body(z_ref, w1, c1, w2, c2, w3, c3, out_ref):
    a = _gelu(jnp.dot(z_ref[...], w1[...], preferred_element_type=_F32)
              + c1[...])
    a = _gelu(jnp.dot(a, w2[...], preferred_element_type=_F32) + c2[...])
    out_ref[...] = jnp.dot(a, w3[...], preferred_element_type=_F32) + c3[...]

  return pl.pallas_call(
      body,
      out_shape=jax.ShapeDtypeStruct((1, 10), _F32),
  )(z, W1, b1, W2, b2, W3, b3)


def kernel(x, edge_index, batch, edge_attr, fields,
           Wl1, bl1, Wr1, Wl2, bl2, Wr2, Wl3, bl3, Wr3,
           p1, p2, p3, W1, b1, W2, b2, W3, b3):
  del batch, edge_attr
  npad = _N_PAD - _N
  h0 = jnp.concatenate([x[:, :3], fields], axis=1)
  hzA = jnp.pad(h0[:, :_HF], ((0, npad), (0, 0)))
  hzB = jnp.pad(h0[:, _HF:], ((0, npad), (0, 0)))
  alive = jnp.pad(jnp.ones((_N,), _F32), (0, npad))

  epad = _E_PAD - _E
  extra = _N + (jnp.arange(epad, dtype=jnp.int32) % npad)
  srcp = jnp.concatenate([edge_index[0].astype(jnp.int32), extra])
  dstp = jnp.concatenate([edge_index[1].astype(jnp.int32), extra])

  tril = jnp.asarray(_TRIL512)
  params = ((Wl1, bl1, Wr1, p1), (Wl2, bl2, Wr2, p2), (Wl3, bl3, Wr3, p3))
  nn = _N
  xs = []
  for lvl in range(3):
    kk = int(math.ceil(0.8 * nn))
    Wl, bl, Wr, p = params[lvl]
    ssumA, ssumB, degA, degB = _sc_segment_sum(hzA, hzB, alive, srcp, dstp)
    hp, sc = _tc_sage(ssumA, ssumB,
                      degA.reshape(_N_PAD, 1), degB.reshape(_N_PAD, 1),
                      hzA, hzB, alive.reshape(_N_PAD, 1),
                      Wl, bl.reshape(1, _F), Wr, p.reshape(1, _F))
    thr, tn = _tc_thresh(sc.reshape(_SROW, 128), kk)
    hzA, hzB, alive1, xl = _tc_select(hp, sc, thr, tn, tril, kk)
    alive = alive1.reshape(_N_PAD)
    xs.append(xl)
    nn = kk

  z = xs[0] + xs[1] + xs[2]
  return _tc_mlp(z, W1, b1.reshape(1, _F), W2, b2.reshape(1, _F),
                 W3, b3.reshape(1, 10))


# speedup vs baseline: 20.0687x; 2.0851x over previous
import functools
import math

import numpy as np
import jax
import jax.numpy as jnp
from jax import lax
from jax.experimental import pallas as pl
from jax.experimental.pallas import tpu as pltpu
from jax.experimental.pallas import tpu_sc as plsc

_N = 50000
_E = 800000
_F = 64
_HF = 32

_NSC = 2
_NTILE = 16
_CH = 128

_N_PAD = 50176
_RPT = _N_PAD // _NTILE
_ZR = 392
_KSUB = 2
_SUP = _KSUB * _CH
_NSUP = 198
_E_PAD = _NTILE * _NSUP * _SUP
_EPT = _E_PAD // _NTILE
_IROW = _EPT // _CH

_BLK = 512
_NBLK = _N_PAD // _BLK
_SROW = _N_PAD // 128

_MIN_I32 = np.int32(-2147483648)
_F32 = jnp.float32

_TRIL512 = np.tril(np.ones((_BLK, _BLK), np.float32), -1)


def _sc_segment_sum(hzA, hzB, alive, srcp2, dstp2, z2d, z1d):
  mesh = plsc.VectorSubcoreMesh(
      core_axis_name="c", subcore_axis_name="s",
      num_cores=_NSC, num_subcores=_NTILE)
  out_type = (
      jax.ShapeDtypeStruct((_N_PAD, _HF), _F32),
      jax.ShapeDtypeStruct((_N_PAD, _HF), _F32),
      jax.ShapeDtypeStruct((_N_PAD,), _F32),
      jax.ShapeDtypeStruct((_N_PAD,), _F32),
  )
  scratch = (
      [pltpu.VMEM((_KSUB, _CH), jnp.int32) for _ in range(3)]
      + [pltpu.VMEM((_KSUB, _CH), jnp.int32) for _ in range(3)]
      + [pltpu.VMEM((_KSUB, _CH, _HF), _F32) for _ in range(3)]
      + [pltpu.VMEM((_KSUB, _CH), _F32) for _ in range(3)]
      + [pltpu.VMEM_SHARED((_N_PAD, _HF), _F32),
         pltpu.VMEM_SHARED((_N_PAD,), _F32)]
      + [pltpu.SemaphoreType.DMA for _ in range(12)]
  )

  @functools.partial(
      pl.kernel, out_type=out_type, mesh=mesh, scratch_types=scratch,
      compiler_params=pltpu.CompilerParams(use_tc_tiling_on_sc=False))
  def k(hzA_h, hzB_h, alive_h, src_h, dst_h, z2d_h, z1d_h,
        ssumA_h, ssumB_h, degA_h, degB_h, *refs):
    srcb = refs[0:3]
    dstb = refs[3:6]
    valb = refs[6:9]
    ab = refs[9:12]
    acc, deg_acc = refs[12:14]
    gsem = refs[14:17]
    ssem = refs[17:20]
    asem = refs[20:23]
    dsem = refs[23:26]
    c = lax.axis_index("c")
    s = lax.axis_index("s")

    row0 = s * _RPT
    pltpu.sync_copy(z2d_h.at[pl.ds(row0, _RPT)], acc.at[pl.ds(row0, _RPT)])
    pltpu.sync_copy(z1d_h.at[pl.ds(row0, _RPT)],
                    deg_acc.at[pl.ds(row0, _RPT)])
    plsc.subcore_barrier()

    irow0 = s * _IROW

    def load(S, slot):
      rb = irow0 + S * _KSUB
      pltpu.sync_copy(src_h.at[pl.ds(rb, _KSUB)], srcb[slot])
      pltpu.sync_copy(dst_h.at[pl.ds(rb, _KSUB)], dstb[slot])

      @pl.when(c == 0)
      def _():
        for kk in range(_KSUB):
          pltpu.async_copy(hzA_h.at[srcb[slot].at[kk]],
                           valb[slot].at[kk], gsem[slot])

      @pl.when(c == 1)
      def _():
        for kk in range(_KSUB):
          pltpu.async_copy(hzB_h.at[srcb[slot].at[kk]],
                           valb[slot].at[kk], gsem[slot])

      @pl.when(S % 2 == c)
      def _():
        for kk in range(_KSUB):
          pltpu.async_copy(alive_h.at[srcb[slot].at[kk]],
                           ab[slot].at[kk], asem[slot])

    def process(S, slot):
      for kk in range(_KSUB):
        pltpu.make_async_copy(hzA_h.at[srcb[slot].at[kk]],
                              valb[slot].at[kk], gsem[slot]).wait()
        pltpu.async_copy(valb[slot].at[kk], acc.at[dstb[slot].at[kk]],
                         ssem[slot], add=True)

      @pl.when(S % 2 == c)
      def _():
        for kk in range(_KSUB):
          pltpu.make_async_copy(alive_h.at[srcb[slot].at[kk]],
                                ab[slot].at[kk], asem[slot]).wait()
          pltpu.async_copy(ab[slot].at[kk], deg_acc.at[dstb[slot].at[kk]],
                           dsem[slot], add=True)

    def drain_scat(S, slot):
      for kk in range(_KSUB):
        pltpu.make_async_copy(valb[slot].at[kk], acc.at[dstb[slot].at[kk]],
                              ssem[slot]).wait()

      @pl.when(S % 2 == c)
      def _():
        for kk in range(_KSUB):
          pltpu.make_async_copy(ab[slot].at[kk],
                                deg_acc.at[dstb[slot].at[kk]],
                                dsem[slot]).wait()

    load(0, 0)
    load(1, 1)

    def body(m, carry):
      for i in range(3):
        S = 3 * m + i
        process(S, i)
        Snxt = S + 2
        slot2 = (i + 2) % 3

        @pl.when(Snxt < _NSUP)
        def _():
          @pl.when(Snxt >= 3)
          def _():
            drain_scat(Snxt - 3, slot2)
          load(Snxt, slot2)
      return carry
    lax.fori_loop(0, _NSUP // 3, body, 0)
    drain_scat(_NSUP - 3, 0)
    drain_scat(_NSUP - 2, 1)
    drain_scat(_NSUP - 1, 2)
    plsc.subcore_barrier()

    @pl.when(c == 0)
    def _():
      pltpu.sync_copy(acc.at[pl.ds(row0, _RPT)],
                      ssumA_h.at[pl.ds(row0, _RPT)])
      pltpu.sync_copy(deg_acc.at[pl.ds(row0, _RPT)],
                      degA_h.at[pl.ds(row0, _RPT)])

    @pl.when(c == 1)
    def _():
      pltpu.sync_copy(acc.at[pl.ds(row0, _RPT)],
                      ssumB_h.at[pl.ds(row0, _RPT)])
      pltpu.sync_copy(deg_acc.at[pl.ds(row0, _RPT)],
                      degB_h.at[pl.ds(row0, _RPT)])

  return k(hzA, hzB, alive, srcp2, dstp2, z2d, z1d)


def _gelu(t):
  return 0.5 * t * (1.0 + lax.erf(t * np.float32(1.0 / math.sqrt(2.0))))


def _tc_sage(ssumA, ssumB, degA, degB, hzA, hzB, alive, Wl, bl, Wr, p):
  def body(sA, sB, dA, dB, hA, hB, al, wl, blv, wr, pv, hp_ref, sc_ref):
    d = jnp.maximum(dA[...] + dB[...], 1.0)
    inv = 1.0 / d
    mean = jnp.concatenate([sA[...], sB[...]], axis=1) * inv
    hz = jnp.concatenate([hA[...], hB[...]], axis=1)
    pre = (jnp.dot(mean, wl[...], preferred_element_type=_F32) + blv[...]
           + jnp.dot(hz, wr[...], preferred_element_type=_F32))
    hp = _gelu(pre)
    pn = pv[...]
    pn = pn * lax.rsqrt(jnp.sum(pn * pn))
    sco = jnp.tanh(jnp.sum(hp * pn, axis=1, keepdims=True))
    hp_ref[...] = hp
    sc_ref[...] = jnp.where(al[...] > 0.0, sco, -2.0)

  nblock = lambda cols: pl.BlockSpec((_BLK, cols), lambda i: (i, 0))
  wblock = lambda r, c: pl.BlockSpec((r, c), lambda i: (0, 0))
  return pl.pallas_call(
      body,
      grid=(_NBLK,),
      in_specs=[nblock(_HF), nblock(_HF), nblock(1), nblock(1),
                nblock(_HF), nblock(_HF), nblock(1),
                wblock(_F, _F), wblock(1, _F), wblock(_F, _F), wblock(1, _F)],
      out_specs=[nblock(_F), nblock(1)],
      out_shape=[jax.ShapeDtypeStruct((_N_PAD, _F), _F32),
                 jax.ShapeDtypeStruct((_N_PAD, 1), _F32)],
  )(ssumA, ssumB, degA, degB, hzA, hzB, alive, Wl, bl, Wr, p)


def _monotone_i32(bits):
  return jnp.where(bits >= 0, bits, bits ^ np.int32(0x7FFFFFFF))


def _tc_thresh(scg, kk):
  def body(sc_ref, thr_ref, tn_ref):
    mi = _monotone_i32(lax.bitcast_convert_type(sc_ref[...], jnp.int32))

    def step(i, P):
      bv = jnp.left_shift(np.int32(1), 31 - i)
      cand = P | bv
      cand_s = cand ^ _MIN_I32
      cnt = jnp.sum((mi >= cand_s).astype(jnp.int32))
      return jnp.where(cnt >= kk, cand, P)

    P = lax.fori_loop(0, 32, step, jnp.int32(0))
    T = P ^ _MIN_I32
    cg = jnp.sum((mi > T).astype(jnp.int32))
    thr_ref[...] = jnp.broadcast_to(T, (1, 1))
    tn_ref[...] = jnp.broadcast_to(kk - cg, (1, 1))

  return pl.pallas_call(
      body,
      out_shape=[jax.ShapeDtypeStruct((1, 1), jnp.int32),
                 jax.ShapeDtypeStruct((1, 1), jnp.int32)],
  )(scg)


def _tc_select(hp, sc, thr, tn, tril, kk):
  def body(hp_ref, sc_ref, thr_ref, tn_ref, tril_ref,
           hzA_ref, hzB_ref, al_ref, x_ref, cnt_ref):
    i = pl.program_id(0)

    @pl.when(i == 0)
    def _():
      cnt_ref[0] = np.int32(0)
      x_ref[...] = jnp.concatenate(
          [jnp.full((1, _F), -1e30, _F32), jnp.zeros((1, _F), _F32)], axis=1)

    scb = sc_ref[...]
    mi = _monotone_i32(lax.bitcast_convert_type(scb, jnp.int32))
    T = thr_ref[...]
    tnf = tn_ref[...].astype(_F32)
    eq = mi == T
    eqf = eq.astype(_F32)
    base = cnt_ref[0].astype(_F32)
    pref = jnp.dot(tril_ref[...], eqf, preferred_element_type=_F32) + base
    sel = (mi > T) | (eq & (pref < tnf))
    cnt_ref[0] = cnt_ref[0] + jnp.sum(eqf).astype(jnp.int32)

    selF = sel.astype(_F32)
    hz = hp_ref[...] * (selF * scb)
    hzA_ref[...] = hz[:, :_HF]
    hzB_ref[...] = hz[:, _HF:]
    al_ref[...] = selF
    pmax = jnp.max(jnp.where(sel, hz, -1e30), axis=0, keepdims=True)
    psum = jnp.sum(hz, axis=0, keepdims=True)
    old = x_ref[...]
    x_ref[...] = jnp.concatenate(
        [jnp.maximum(old[:, :_F], pmax), old[:, _F:] + psum], axis=1)

    @pl.when(i == _NBLK - 1)
    def _():
      fin = x_ref[...]
      x_ref[...] = jnp.concatenate(
          [fin[:, :_F], fin[:, _F:] * np.float32(1.0 / kk)], axis=1)

  nblock = lambda cols: pl.BlockSpec((_BLK, cols), lambda i: (i, 0))
  full = lambda r, c: pl.BlockSpec((r, c), lambda i: (0, 0))
  return pl.pallas_call(
      body,
      grid=(_NBLK,),
      in_specs=[nblock(_F), nblock(1), full(1, 1), full(1, 1),
                full(_BLK, _BLK)],
      out_specs=[nblock(_HF), nblock(_HF), nblock(1), full(1, 2 * _F)],
      out_shape=[jax.ShapeDtypeStruct((_N_PAD, _HF), _F32),
                 jax.ShapeDtypeStruct((_N_PAD, _HF), _F32),
                 jax.ShapeDtypeStruct((_N_PAD, 1), _F32),
                 jax.ShapeDtypeStruct((1, 2 * _F), _F32)],
      scratch_shapes=[pltpu.SMEM((1,), jnp.int32)],
  )(hp, sc, thr, tn, tril)


def _tc_mlp(z, W1, b1, W2, b2, W3, b3):
  def body(z_ref, w1, c1, w2, c2, w3, c3, out_ref):
    a = _gelu(jnp.dot(z_ref[...], w1[...], preferred_element_type=_F32)
              + c1[...])
    a = _gelu(jnp.dot(a, w2[...], preferred_element_type=_F32) + c2[...])
    out_ref[...] = jnp.dot(a, w3[...], preferred_element_type=_F32) + c3[...]

  return pl.pallas_call(
      body,
      out_shape=jax.ShapeDtypeStruct((1, 10), _F32),
  )(z, W1, b1, W2, b2, W3, b3)


def kernel(x, edge_index, batch, edge_attr, fields,
           Wl1, bl1, Wr1, Wl2, bl2, Wr2, Wl3, bl3, Wr3,
           p1, p2, p3, W1, b1, W2, b2, W3, b3):
  del batch, edge_attr
  npad = _N_PAD - _N
  h0 = jnp.concatenate([x[:, :3], fields], axis=1)
  hzA = jnp.pad(h0[:, :_HF], ((0, npad), (0, 0)))
  hzB = jnp.pad(h0[:, _HF:], ((0, npad), (0, 0)))
  alive = jnp.pad(jnp.ones((_N,), _F32), (0, npad))

  epad = _E_PAD - _E
  extra = _N + (jnp.arange(epad, dtype=jnp.int32) % npad)
  srcp = jnp.concatenate([edge_index[0].astype(jnp.int32),
                          extra]).reshape(_E_PAD // _CH, _CH)
  dstp = jnp.concatenate([edge_index[1].astype(jnp.int32),
                          extra]).reshape(_E_PAD // _CH, _CH)

  tril = jnp.asarray(_TRIL512)
  z2d = jnp.zeros((_N_PAD, _HF), _F32)
  z1d = jnp.zeros((_N_PAD,), _F32)
  params = ((Wl1, bl1, Wr1, p1), (Wl2, bl2, Wr2, p2), (Wl3, bl3, Wr3, p3))
  nn = _N
  xs = []
  for lvl in range(3):
    kk = int(math.ceil(0.8 * nn))
    Wl, bl, Wr, p = params[lvl]
    ssumA, ssumB, degA, degB = _sc_segment_sum(hzA, hzB, alive, srcp, dstp,
                                               z2d, z1d)
    hp, sc = _tc_sage(ssumA, ssumB,
                      degA.reshape(_N_PAD, 1), degB.reshape(_N_PAD, 1),
                      hzA, hzB, alive.reshape(_N_PAD, 1),
                      Wl, bl.reshape(1, _F), Wr, p.reshape(1, _F))
    thr, tn = _tc_thresh(sc.reshape(_SROW, 128), kk)
    hzA, hzB, alive1, xl = _tc_select(hp, sc, thr, tn, tril, kk)
    alive = alive1.reshape(_N_PAD)
    xs.append(xl)
    nn = kk

  z = xs[0] + xs[1] + xs[2]
  return _tc_mlp(z, W1, b1.reshape(1, _F), W2, b2.reshape(1, _F),
                 W3, b3.reshape(1, 10))


# speedup vs baseline: 24.9514x; 1.2433x over previous
import functools
import math

import numpy as np
import jax
import jax.numpy as jnp
from jax import lax
from jax.experimental import pallas as pl
from jax.experimental.pallas import tpu as pltpu
from jax.experimental.pallas import tpu_sc as plsc

_N = 50000
_E = 800000
_F = 64
_HF = 32

_NSC = 2
_NTILE = 16
_CH = 128

_N_PAD = 50176
_RPT = _N_PAD // _NTILE
_ZR = 392
_KSUB = 2
_SUP = _KSUB * _CH
_NSUP = 198
_E_PAD = _NTILE * _NSUP * _SUP
_EPT = _E_PAD // _NTILE
_IROW = _EPT // _CH

_BLK = 1024
_NBLK = _N_PAD // _BLK
_SROW = _N_PAD // 128

_MIN_I32 = np.int32(-2147483648)
_F32 = jnp.float32

_TRIU128 = np.triu(np.ones((128, 128), np.float32), 1)


def _sc_segment_sum(hzA, hzB, alive, srcp2, dstp2, z2d, z1d):
  mesh = plsc.VectorSubcoreMesh(
      core_axis_name="c", subcore_axis_name="s",
      num_cores=_NSC, num_subcores=_NTILE)
  out_type = (
      jax.ShapeDtypeStruct((_N_PAD, _HF), _F32),
      jax.ShapeDtypeStruct((_N_PAD, _HF), _F32),
      jax.ShapeDtypeStruct((_N_PAD,), _F32),
      jax.ShapeDtypeStruct((_N_PAD,), _F32),
  )
  scratch = (
      [pltpu.VMEM((_KSUB, _CH), jnp.int32) for _ in range(3)]
      + [pltpu.VMEM((_KSUB, _CH), jnp.int32) for _ in range(3)]
      + [pltpu.VMEM((_KSUB, _CH, _HF), _F32) for _ in range(3)]
      + [pltpu.VMEM((_KSUB, _CH), _F32) for _ in range(3)]
      + [pltpu.VMEM_SHARED((_N_PAD, _HF), _F32),
         pltpu.VMEM_SHARED((_N_PAD,), _F32)]
      + [pltpu.SemaphoreType.DMA for _ in range(12)]
  )

  @functools.partial(
      pl.kernel, out_type=out_type, mesh=mesh, scratch_types=scratch,
      compiler_params=pltpu.CompilerParams(use_tc_tiling_on_sc=False))
  def k(hzA_h, hzB_h, alive_h, src_h, dst_h, z2d_h, z1d_h,
        ssumA_h, ssumB_h, degA_h, degB_h, *refs):
    srcb = refs[0:3]
    dstb = refs[3:6]
    valb = refs[6:9]
    ab = refs[9:12]
    acc, deg_acc = refs[12:14]
    gsem = refs[14:17]
    ssem = refs[17:20]
    asem = refs[20:23]
    dsem = refs[23:26]
    c = lax.axis_index("c")
    s = lax.axis_index("s")

    row0 = s * _RPT
    pltpu.sync_copy(z2d_h.at[pl.ds(row0, _RPT)], acc.at[pl.ds(row0, _RPT)])
    pltpu.sync_copy(z1d_h.at[pl.ds(row0, _RPT)],
                    deg_acc.at[pl.ds(row0, _RPT)])
    plsc.subcore_barrier()

    irow0 = s * _IROW

    def load(S, slot):
      rb = irow0 + S * _KSUB
      pltpu.sync_copy(src_h.at[pl.ds(rb, _KSUB)], srcb[slot])
      pltpu.sync_copy(dst_h.at[pl.ds(rb, _KSUB)], dstb[slot])

      @pl.when(c == 0)
      def _():
        for kk in range(_KSUB):
          pltpu.async_copy(hzA_h.at[srcb[slot].at[kk]],
                           valb[slot].at[kk], gsem[slot])

      @pl.when(c == 1)
      def _():
        for kk in range(_KSUB):
          pltpu.async_copy(hzB_h.at[srcb[slot].at[kk]],
                           valb[slot].at[kk], gsem[slot])

      @pl.when(S % 2 == c)
      def _():
        for kk in range(_KSUB):
          pltpu.async_copy(alive_h.at[srcb[slot].at[kk]],
                           ab[slot].at[kk], asem[slot])

    def process(S, slot):
      for kk in range(_KSUB):
        pltpu.make_async_copy(hzA_h.at[srcb[slot].at[kk]],
                              valb[slot].at[kk], gsem[slot]).wait()
        pltpu.async_copy(valb[slot].at[kk], acc.at[dstb[slot].at[kk]],
                         ssem[slot], add=True)

      @pl.when(S % 2 == c)
      def _():
        for kk in range(_KSUB):
          pltpu.make_async_copy(alive_h.at[srcb[slot].at[kk]],
                                ab[slot].at[kk], asem[slot]).wait()
          pltpu.async_copy(ab[slot].at[kk], deg_acc.at[dstb[slot].at[kk]],
                           dsem[slot], add=True)

    def drain_scat(S, slot):
      for kk in range(_KSUB):
        pltpu.make_async_copy(valb[slot].at[kk], acc.at[dstb[slot].at[kk]],
                              ssem[slot]).wait()

      @pl.when(S % 2 == c)
      def _():
        for kk in range(_KSUB):
          pltpu.make_async_copy(ab[slot].at[kk],
                                deg_acc.at[dstb[slot].at[kk]],
                                dsem[slot]).wait()

    load(0, 0)
    load(1, 1)

    def body(m, carry):
      for i in range(3):
        S = 3 * m + i
        process(S, i)
        Snxt = S + 2
        slot2 = (i + 2) % 3

        @pl.when(Snxt < _NSUP)
        def _():
          @pl.when(Snxt >= 3)
          def _():
            drain_scat(Snxt - 3, slot2)
          load(Snxt, slot2)
      return carry
    lax.fori_loop(0, _NSUP // 3, body, 0)
    drain_scat(_NSUP - 3, 0)
    drain_scat(_NSUP - 2, 1)
    drain_scat(_NSUP - 1, 2)
    plsc.subcore_barrier()

    @pl.when(c == 0)
    def _():
      pltpu.sync_copy(acc.at[pl.ds(row0, _RPT)],
                      ssumA_h.at[pl.ds(row0, _RPT)])
      pltpu.sync_copy(deg_acc.at[pl.ds(row0, _RPT)],
                      degA_h.at[pl.ds(row0, _RPT)])

    @pl.when(c == 1)
    def _():
      pltpu.sync_copy(acc.at[pl.ds(row0, _RPT)],
                      ssumB_h.at[pl.ds(row0, _RPT)])
      pltpu.sync_copy(deg_acc.at[pl.ds(row0, _RPT)],
                      degB_h.at[pl.ds(row0, _RPT)])

  return k(hzA, hzB, alive, srcp2, dstp2, z2d, z1d)


def _gelu(t):
  return 0.5 * t * (1.0 + lax.erf(t * np.float32(1.0 / math.sqrt(2.0))))


def _tc_sage(ssumA, ssumB, degA, degB, hzA, hzB, alive, Wl, bl, Wr, p):
  def body(sA, sB, dA, dB, hA, hB, al, wl, blv, wr, pv, hp_ref, sc_ref):
    d = jnp.maximum(dA[...] + dB[...], 1.0)
    inv = jnp.transpose((1.0 / d).reshape(1, _BLK))
    mean = jnp.concatenate([sA[...], sB[...]], axis=1) * inv
    hz = jnp.concatenate([hA[...], hB[...]], axis=1)
    pre = (jnp.dot(mean, wl[...], preferred_element_type=_F32) + blv[...]
           + jnp.dot(hz, wr[...], preferred_element_type=_F32))
    hp = _gelu(pre)
    pn = pv[...]
    pn = pn * lax.rsqrt(jnp.sum(pn * pn))
    sco = jnp.tanh(jnp.sum(hp * pn, axis=1, keepdims=True))
    hp_ref[...] = hp
    sc_row = jnp.transpose(sco).reshape(_BLK // 128, 128)
    sc_ref[...] = jnp.where(al[...] > 0.0, sc_row, -2.0)

  nblock = lambda cols: pl.BlockSpec((_BLK, cols), lambda i: (i, 0))
  cblock = pl.BlockSpec((_BLK // 128, 128), lambda i: (i, 0))
  wblock = lambda r, c: pl.BlockSpec((r, c), lambda i: (0, 0))
  return pl.pallas_call(
      body,
      grid=(_NBLK,),
      in_specs=[nblock(_HF), nblock(_HF), cblock, cblock,
                nblock(_HF), nblock(_HF), cblock,
                wblock(_F, _F), wblock(1, _F), wblock(_F, _F), wblock(1, _F)],
      out_specs=[nblock(_F), cblock],
      out_shape=[jax.ShapeDtypeStruct((_N_PAD, _F), _F32),
                 jax.ShapeDtypeStruct((_SROW, 128), _F32)],
  )(ssumA, ssumB, degA, degB, hzA, hzB, alive, Wl, bl, Wr, p)


def _monotone_i32(bits):
  return jnp.where(bits >= 0, bits, bits ^ np.int32(0x7FFFFFFF))


def _tc_thresh(scg, kk):
  def body(sc_ref, thr_ref, tn_ref):
    mi = _monotone_i32(lax.bitcast_convert_type(sc_ref[...], jnp.int32))

    def step(i, P):
      bv = jnp.left_shift(np.int32(1), 31 - i)
      cand = P | bv
      cand_s = cand ^ _MIN_I32
      cnt = jnp.sum((mi >= cand_s).astype(jnp.int32))
      return jnp.where(cnt >= kk, cand, P)

    P = lax.fori_loop(0, 32, step, jnp.int32(0))
    T = P ^ _MIN_I32
    cg = jnp.sum((mi > T).astype(jnp.int32))
    thr_ref[...] = jnp.broadcast_to(T, (1, 1))
    tn_ref[...] = jnp.broadcast_to(kk - cg, (1, 1))

  return pl.pallas_call(
      body,
      out_shape=[jax.ShapeDtypeStruct((1, 1), jnp.int32),
                 jax.ShapeDtypeStruct((1, 1), jnp.int32)],
  )(scg)


def _tc_select(hp, sc, thr, tn, triu, kk):
  nrow = _BLK // 128

  def body(hp_ref, sc_ref, thr_ref, tn_ref, u_ref,
           hzA_ref, hzB_ref, al_ref, x_ref, cnt_ref):
    i = pl.program_id(0)

    @pl.when(i == 0)
    def _():
      cnt_ref[0] = np.int32(0)
      x_ref[...] = jnp.concatenate(
          [jnp.full((1, _F), -1e30, _F32), jnp.zeros((1, _F), _F32)], axis=1)

    scb = sc_ref[...]
    mi = _monotone_i32(lax.bitcast_convert_type(scb, jnp.int32))
    T = thr_ref[...]
    tnf = tn_ref[...].astype(_F32)
    eq = mi == T
    eqf = eq.astype(_F32)
    rows = jnp.sum(eqf, axis=1, keepdims=True)
    ri = lax.broadcasted_iota(jnp.int32, (nrow, nrow), 0)
    ci = lax.broadcasted_iota(jnp.int32, (nrow, nrow), 1)
    ls = (ri > ci).astype(_F32)
    rowpref = jnp.dot(ls, rows, preferred_element_type=_F32)
    inrow = jnp.dot(eqf, u_ref[...], preferred_element_type=_F32)
    base = cnt_ref[0].astype(_F32)
    pref = rowpref + inrow + base
    sel = (mi > T) | (eq & (pref < tnf))
    cnt_ref[0] = cnt_ref[0] + jnp.sum(eqf).astype(jnp.int32)

    selF = sel.astype(_F32)
    fac = jnp.transpose((selF * scb).reshape(1, _BLK))
    s512 = jnp.transpose(selF.reshape(1, _BLK))
    hz = hp_ref[...] * fac
    hzA_ref[...] = hz[:, :_HF]
    hzB_ref[...] = hz[:, _HF:]
    al_ref[...] = selF
    pmax = jnp.max(jnp.where(s512 > 0.0, hz, -1e30), axis=0, keepdims=True)
    psum = jnp.sum(hz, axis=0, keepdims=True)
    old = x_ref[...]
    x_ref[...] = jnp.concatenate(
        [jnp.maximum(old[:, :_F], pmax), old[:, _F:] + psum], axis=1)

    @pl.when(i == _NBLK - 1)
    def _():
      fin = x_ref[...]
      x_ref[...] = jnp.concatenate(
          [fin[:, :_F], fin[:, _F:] * np.float32(1.0 / kk)], axis=1)

  nblock = lambda cols: pl.BlockSpec((_BLK, cols), lambda i: (i, 0))
  cblock = pl.BlockSpec((nrow, 128), lambda i: (i, 0))
  full = lambda r, c: pl.BlockSpec((r, c), lambda i: (0, 0))
  return pl.pallas_call(
      body,
      grid=(_NBLK,),
      in_specs=[nblock(_F), cblock, full(1, 1), full(1, 1),
                full(128, 128)],
      out_specs=[nblock(_HF), nblock(_HF), cblock, full(1, 2 * _F)],
      out_shape=[jax.ShapeDtypeStruct((_N_PAD, _HF), _F32),
                 jax.ShapeDtypeStruct((_N_PAD, _HF), _F32),
                 jax.ShapeDtypeStruct((_SROW, 128), _F32),
                 jax.ShapeDtypeStruct((1, 2 * _F), _F32)],
      scratch_shapes=[pltpu.SMEM((1,), jnp.int32)],
  )(hp, sc, thr, tn, triu)


def _tc_mlp(z, W1, b1, W2, b2, W3, b3):
  def body(z_ref, w1, c1, w2, c2, w3, c3, out_ref):
    a = _gelu(jnp.dot(z_ref[...], w1[...], preferred_element_type=_F32)
              + c1[...])
    a = _gelu(jnp.dot(a, w2[...], preferred_element_type=_F32) + c2[...])
    out_ref[...] = jnp.dot(a, w3[...], preferred_element_type=_F32) + c3[...]

  return pl.pallas_call(
      body,
      out_shape=jax.ShapeDtypeStruct((1, 10), _F32),
  )(z, W1, b1, W2, b2, W3, b3)


def kernel(x, edge_index, batch, edge_attr, fields,
           Wl1, bl1, Wr1, Wl2, bl2, Wr2, Wl3, bl3, Wr3,
           p1, p2, p3, W1, b1, W2, b2, W3, b3):
  del batch, edge_attr
  npad = _N_PAD - _N
  h0 = jnp.concatenate([x[:, :3], fields], axis=1)
  hzA = jnp.pad(h0[:, :_HF], ((0, npad), (0, 0)))
  hzB = jnp.pad(h0[:, _HF:], ((0, npad), (0, 0)))
  alive = jnp.pad(jnp.ones((_N,), _F32), (0, npad))

  epad = _E_PAD - _E
  extra = _N + (jnp.arange(epad, dtype=jnp.int32) % npad)
  srcp = jnp.concatenate([edge_index[0].astype(jnp.int32),
                          extra]).reshape(_E_PAD // _CH, _CH)
  dstp = jnp.concatenate([edge_index[1].astype(jnp.int32),
                          extra]).reshape(_E_PAD // _CH, _CH)

  triu = jnp.asarray(_TRIU128)
  z2d = jnp.zeros((_N_PAD, _HF), _F32)
  z1d = jnp.zeros((_N_PAD,), _F32)
  params = ((Wl1, bl1, Wr1, p1), (Wl2, bl2, Wr2, p2), (Wl3, bl3, Wr3, p3))
  nn = _N
  xs = []
  for lvl in range(3):
    kk = int(math.ceil(0.8 * nn))
    Wl, bl, Wr, p = params[lvl]
    ssumA, ssumB, degA, degB = _sc_segment_sum(hzA, hzB, alive, srcp, dstp,
                                               z2d, z1d)
    hp, sc = _tc_sage(ssumA, ssumB,
                      degA.reshape(_SROW, 128), degB.reshape(_SROW, 128),
                      hzA, hzB, alive.reshape(_SROW, 128),
                      Wl, bl.reshape(1, _F), Wr, p.reshape(1, _F))
    thr, tn = _tc_thresh(sc, kk)
    hzA, hzB, alive1, xl = _tc_select(hp, sc, thr, tn, triu, kk)
    alive = alive1.reshape(_N_PAD)
    xs.append(xl)
    nn = kk

  z = xs[0] + xs[1] + xs[2]
  return _tc_mlp(z, W1, b1.reshape(1, _F), W2, b2.reshape(1, _F),
                 W3, b3.reshape(1, 10))


# speedup vs baseline: 31.1747x; 1.2494x over previous
import functools
import math

import numpy as np
import jax
import jax.numpy as jnp
from jax import lax
from jax.experimental import pallas as pl
from jax.experimental.pallas import tpu as pltpu
from jax.experimental.pallas import tpu_sc as plsc

_N = 50000
_E = 800000
_F = 64
_HF = 32

_NSC = 2
_NTILE = 16
_CH = 128

_N_PAD = 50176
_RPT = _N_PAD // _NTILE
_ZR = 392
_KSUB = 2
_SUP = _KSUB * _CH
_NSUP = 198
_E_PAD = _NTILE * _NSUP * _SUP
_EPT = _E_PAD // _NTILE
_IROW = _EPT // _CH

_BLK = 1024
_NBLK = _N_PAD // _BLK
_SROW = _N_PAD // 128

_MIN_I32 = np.int32(-2147483648)
_F32 = jnp.float32

_TRIU128 = np.triu(np.ones((128, 128), np.float32), 1)


def _sc_segment_sum(hz4, alive, combA, combB, z2d, z1d):
  mesh = plsc.VectorSubcoreMesh(
      core_axis_name="c", subcore_axis_name="s",
      num_cores=_NSC, num_subcores=_NTILE)
  out_type = (
      jax.ShapeDtypeStruct((_N_PAD, 128), _F32),
      jax.ShapeDtypeStruct((_N_PAD,), _F32),
      jax.ShapeDtypeStruct((_N_PAD,), _F32),
  )
  scratch = (
      [pltpu.VMEM((2 * _KSUB, _CH), jnp.int32) for _ in range(3)]
      + [pltpu.VMEM((_KSUB, _CH, _HF), _F32) for _ in range(3)]
      + [pltpu.VMEM((_KSUB, _CH), _F32) for _ in range(3)]
      + [pltpu.VMEM_SHARED((_N_PAD, _HF), _F32),
         pltpu.VMEM_SHARED((_N_PAD,), _F32)]
      + [pltpu.SemaphoreType.DMA for _ in range(12)]
  )

  @functools.partial(
      pl.kernel, out_type=out_type, mesh=mesh, scratch_types=scratch,
      compiler_params=pltpu.CompilerParams(use_tc_tiling_on_sc=False))
  def k(hz4_h, alive_h, combA_h, combB_h, z2d_h, z1d_h,
        ssum_h, degA_h, degB_h, *refs):
    ib = refs[0:3]
    valb = refs[3:6]
    ab = refs[6:9]
    acc, deg_acc = refs[9:11]
    gsem = refs[11:14]
    ssem = refs[14:17]
    asem = refs[17:20]
    dsem = refs[20:23]
    c = lax.axis_index("c")
    s = lax.axis_index("s")

    row0 = s * _RPT
    pltpu.sync_copy(z2d_h.at[pl.ds(row0, _RPT)], acc.at[pl.ds(row0, _RPT)])
    pltpu.sync_copy(z1d_h.at[pl.ds(row0, _RPT)],
                    deg_acc.at[pl.ds(row0, _RPT)])
    plsc.subcore_barrier()

    grp0 = s * _NSUP

    def load(S, slot):
      rb = (grp0 + S) * 2 * _KSUB

      @pl.when(c == 0)
      def _():
        pltpu.sync_copy(combA_h.at[pl.ds(rb, 2 * _KSUB)], ib[slot])

      @pl.when(c == 1)
      def _():
        pltpu.sync_copy(combB_h.at[pl.ds(rb, 2 * _KSUB)], ib[slot])

      for kk in range(_KSUB):
        pltpu.async_copy(hz4_h.at[ib[slot].at[kk]],
                         valb[slot].at[kk], gsem[slot])

      @pl.when(S % 2 == c)
      def _():
        for kk in range(_KSUB):
          pltpu.async_copy(alive_h.at[ib[slot].at[kk]],
                           ab[slot].at[kk], asem[slot])

    def process(S, slot):
      for kk in range(_KSUB):
        pltpu.make_async_copy(hz4_h.at[ib[slot].at[kk]],
                              valb[slot].at[kk], gsem[slot]).wait()
        pltpu.async_copy(valb[slot].at[kk], acc.at[ib[slot].at[_KSUB + kk]],
                         ssem[slot], add=True)

      @pl.when(S % 2 == c)
      def _():
        for kk in range(_KSUB):
          pltpu.make_async_copy(alive_h.at[ib[slot].at[kk]],
                                ab[slot].at[kk], asem[slot]).wait()
          pltpu.async_copy(ab[slot].at[kk],
                           deg_acc.at[ib[slot].at[_KSUB + kk]],
                           dsem[slot], add=True)

    def drain_scat(S, slot):
      for kk in range(_KSUB):
        pltpu.make_async_copy(valb[slot].at[kk],
                              acc.at[ib[slot].at[_KSUB + kk]],
                              ssem[slot]).wait()

      @pl.when(S % 2 == c)
      def _():
        for kk in range(_KSUB):
          pltpu.make_async_copy(ab[slot].at[kk],
                                deg_acc.at[ib[slot].at[_KSUB + kk]],
                                dsem[slot]).wait()

    load(0, 0)
    load(1, 1)

    def body(m, carry):
      for i in range(3):
        S = 3 * m + i
        process(S, i)
        Snxt = S + 2
        slot2 = (i + 2) % 3

        @pl.when(Snxt < _NSUP)
        def _():
          @pl.when(Snxt >= 3)
          def _():
            drain_scat(Snxt - 3, slot2)
          load(Snxt, slot2)
      return carry
    lax.fori_loop(0, _NSUP // 3, body, 0)
    drain_scat(_NSUP - 3, 0)
    drain_scat(_NSUP - 2, 1)
    drain_scat(_NSUP - 1, 2)
    plsc.subcore_barrier()

    @pl.when(c == 0)
    def _():
      pltpu.sync_copy(acc.at[pl.ds(row0, _RPT)],
                      ssum_h.at[pl.ds(row0, _RPT), pl.ds(0, _HF)])
      pltpu.sync_copy(deg_acc.at[pl.ds(row0, _RPT)],
                      degA_h.at[pl.ds(row0, _RPT)])

    @pl.when(c == 1)
    def _():
      pltpu.sync_copy(acc.at[pl.ds(row0, _RPT)],
                      ssum_h.at[pl.ds(row0, _RPT), pl.ds(_HF, _HF)])
      pltpu.sync_copy(deg_acc.at[pl.ds(row0, _RPT)],
                      degB_h.at[pl.ds(row0, _RPT)])

  return k(hz4, alive, combA, combB, z2d, z1d)


def _gelu(t):
  return 0.5 * t * (1.0 + lax.erf(t * np.float32(1.0 / math.sqrt(2.0))))


def _tc_sage(ssum, degA, degB, hz128, alive, Wl, bl, Wr, p):
  def body(ss, dA, dB, hzb, al, wl, blv, wr, pv, hp_ref, sc_ref):
    d = jnp.maximum(dA[...] + dB[...], 1.0)
    inv = jnp.transpose((1.0 / d).reshape(1, _BLK))
    mean = ss[...][:, :_F] * inv
    hz = hzb[...][:, :_F]
    pre = (jnp.dot(mean, wl[...], preferred_element_type=_F32) + blv[...]
           + jnp.dot(hz, wr[...], preferred_element_type=_F32))
    hp = _gelu(pre)
    pn = pv[...]
    pn = pn * lax.rsqrt(jnp.sum(pn * pn))
    sco = jnp.tanh(jnp.sum(hp * pn, axis=1, keepdims=True))
    hp_ref[...] = hp
    sc_row = jnp.transpose(sco).reshape(_BLK // 128, 128)
    sc_ref[...] = jnp.where(al[...] > 0.0, sc_row, -2.0)

  nblock = lambda cols: pl.BlockSpec((_BLK, cols), lambda i: (i, 0))
  cblock = pl.BlockSpec((_BLK // 128, 128), lambda i: (i, 0))
  wblock = lambda r, c: pl.BlockSpec((r, c), lambda i: (0, 0))
  return pl.pallas_call(
      body,
      grid=(_NBLK,),
      in_specs=[nblock(128), cblock, cblock, nblock(128), cblock,
                wblock(_F, _F), wblock(1, _F), wblock(_F, _F), wblock(1, _F)],
      out_specs=[nblock(_F), cblock],
      out_shape=[jax.ShapeDtypeStruct((_N_PAD, _F), _F32),
                 jax.ShapeDtypeStruct((_SROW, 128), _F32)],
  )(ssum, degA, degB, hz128, alive, Wl, bl, Wr, p)


def _monotone_i32(bits):
  return jnp.where(bits >= 0, bits, bits ^ np.int32(0x7FFFFFFF))


def _tc_thresh(scg, kk):
  def body(sc_ref, thr_ref, tn_ref):
    mi = _monotone_i32(lax.bitcast_convert_type(sc_ref[...], jnp.int32))

    def step(i, P):
      bv = jnp.left_shift(np.int32(1), 31 - i)
      cand = P | bv
      cand_s = cand ^ _MIN_I32
      cnt = jnp.sum((mi >= cand_s).astype(jnp.int32))
      return jnp.where(cnt >= kk, cand, P)

    P = lax.fori_loop(0, 32, step, jnp.int32(0))
    T = P ^ _MIN_I32
    cg = jnp.sum((mi > T).astype(jnp.int32))
    thr_ref[...] = jnp.broadcast_to(T, (1, 1))
    tn_ref[...] = jnp.broadcast_to(kk - cg, (1, 1))

  return pl.pallas_call(
      body,
      out_shape=[jax.ShapeDtypeStruct((1, 1), jnp.int32),
                 jax.ShapeDtypeStruct((1, 1), jnp.int32)],
  )(scg)


def _tc_select(hp, sc, thr, tn, triu, kk):
  nrow = _BLK // 128

  def body(hp_ref, sc_ref, thr_ref, tn_ref, u_ref,
           hz_ref, al_ref, x_ref, cnt_ref):
    i = pl.program_id(0)

    @pl.when(i == 0)
    def _():
      cnt_ref[0] = np.int32(0)
      x_ref[...] = jnp.concatenate(
          [jnp.full((1, _F), -1e30, _F32), jnp.zeros((1, _F), _F32)], axis=1)

    scb = sc_ref[...]
    mi = _monotone_i32(lax.bitcast_convert_type(scb, jnp.int32))
    T = thr_ref[...]
    tnf = tn_ref[...].astype(_F32)
    eq = mi == T
    eqf = eq.astype(_F32)
    rows = jnp.sum(eqf, axis=1, keepdims=True)
    ri = lax.broadcasted_iota(jnp.int32, (nrow, nrow), 0)
    ci = lax.broadcasted_iota(jnp.int32, (nrow, nrow), 1)
    ls = (ri > ci).astype(_F32)
    rowpref = jnp.dot(ls, rows, preferred_element_type=_F32)
    inrow = jnp.dot(eqf, u_ref[...], preferred_element_type=_F32)
    base = cnt_ref[0].astype(_F32)
    pref = rowpref + inrow + base
    sel = (mi > T) | (eq & (pref < tnf))
    cnt_ref[0] = cnt_ref[0] + jnp.sum(eqf).astype(jnp.int32)

    selF = sel.astype(_F32)
    fac = jnp.transpose((selF * scb).reshape(1, _BLK))
    s512 = jnp.transpose(selF.reshape(1, _BLK))
    hz = hp_ref[...] * fac
    hz_ref[...] = jnp.concatenate(
        [hz, jnp.zeros((_BLK, 128 - _F), _F32)], axis=1)
    al_ref[...] = selF
    pmax = jnp.max(jnp.where(s512 > 0.0, hz, -1e30), axis=0, keepdims=True)
    psum = jnp.sum(hz, axis=0, keepdims=True)
    old = x_ref[...]
    x_ref[...] = jnp.concatenate(
        [jnp.maximum(old[:, :_F], pmax), old[:, _F:] + psum], axis=1)

    @pl.when(i == _NBLK - 1)
    def _():
      fin = x_ref[...]
      x_ref[...] = jnp.concatenate(
          [fin[:, :_F], fin[:, _F:] * np.float32(1.0 / kk)], axis=1)

  nblock = lambda cols: pl.BlockSpec((_BLK, cols), lambda i: (i, 0))
  cblock = pl.BlockSpec((nrow, 128), lambda i: (i, 0))
  full = lambda r, c: pl.BlockSpec((r, c), lambda i: (0, 0))
  return pl.pallas_call(
      body,
      grid=(_NBLK,),
      in_specs=[nblock(_F), cblock, full(1, 1), full(1, 1),
                full(128, 128)],
      out_specs=[nblock(128), cblock, full(1, 2 * _F)],
      out_shape=[jax.ShapeDtypeStruct((_N_PAD, 128), _F32),
                 jax.ShapeDtypeStruct((_SROW, 128), _F32),
                 jax.ShapeDtypeStruct((1, 2 * _F), _F32)],
      scratch_shapes=[pltpu.SMEM((1,), jnp.int32)],
  )(hp, sc, thr, tn, triu)


def _tc_mlp(z, W1, b1, W2, b2, W3, b3):
  def body(z_ref, w1, c1, w2, c2, w3, c3, out_ref):
    a = _gelu(jnp.dot(z_ref[...], w1[...], preferred_element_type=_F32)
              + c1[...])
    a = _gelu(jnp.dot(a, w2[...], preferred_element_type=_F32) + c2[...])
    out_ref[...] = jnp.dot(a, w3[...], preferred_element_type=_F32) + c3[...]

  return pl.pallas_call(
      body,
      out_shape=jax.ShapeDtypeStruct((1, 10), _F32),
  )(z, W1, b1, W2, b2, W3, b3)


def kernel(x, edge_index, batch, edge_attr, fields,
           Wl1, bl1, Wr1, Wl2, bl2, Wr2, Wl3, bl3, Wr3,
           p1, p2, p3, W1, b1, W2, b2, W3, b3):
  del batch, edge_attr
  npad = _N_PAD - _N
  h0 = jnp.concatenate([x[:, :3], fields], axis=1)
  hz128 = jnp.pad(h0, ((0, npad), (0, 128 - _F)))
  alive = jnp.pad(jnp.ones((_N,), _F32), (0, npad))

  epad = _E_PAD - _E
  extra = _N + (jnp.arange(epad, dtype=jnp.int32) % npad)
  src = jnp.concatenate([edge_index[0].astype(jnp.int32), extra])
  dst = jnp.concatenate([edge_index[1].astype(jnp.int32), extra])
  s3 = src.reshape(_NTILE * _NSUP, _KSUB, _CH)
  d3 = dst.reshape(_NTILE * _NSUP, _KSUB, _CH)
  combA = jnp.concatenate([4 * s3, d3], axis=1).reshape(-1, _CH)
  combB = jnp.concatenate([4 * s3 + 1, d3], axis=1).reshape(-1, _CH)

  triu = jnp.asarray(_TRIU128)
  z2d = jnp.zeros((_N_PAD, _HF), _F32)
  z1d = jnp.zeros((_N_PAD,), _F32)
  params = ((Wl1, bl1, Wr1, p1), (Wl2, bl2, Wr2, p2), (Wl3, bl3, Wr3, p3))
  nn = _N
  xs = []
  for lvl in range(3):
    kk = int(math.ceil(0.8 * nn))
    Wl, bl, Wr, p = params[lvl]
    hz4 = hz128.reshape(4 * _N_PAD, _HF)
    alive4 = jnp.repeat(alive, 4)
    ssum, degA, degB = _sc_segment_sum(hz4, alive4, combA, combB, z2d, z1d)
    hp, sc = _tc_sage(ssum,
                      degA.reshape(_SROW, 128), degB.reshape(_SROW, 128),
                      hz128, alive.reshape(_SROW, 128),
                      Wl, bl.reshape(1, _F), Wr, p.reshape(1, _F))
    thr, tn = _tc_thresh(sc, kk)
    hz128, alive1, xl = _tc_select(hp, sc, thr, tn, triu, kk)
    alive = alive1.reshape(_N_PAD)
    xs.append(xl)
    nn = kk

  z = xs[0] + xs[1] + xs[2]
  return _tc_mlp(z, W1, b1.reshape(1, _F), W2, b2.reshape(1, _F),
                 W3, b3.reshape(1, 10))


# speedup vs baseline: 33.4174x; 1.0719x over previous
import functools
import math

import numpy as np
import jax
import jax.numpy as jnp
from jax import lax
from jax.experimental import pallas as pl
from jax.experimental.pallas import tpu as pltpu
from jax.experimental.pallas import tpu_sc as plsc

_N = 50000
_E = 800000
_F = 64
_HF = 32

_NSC = 2
_NTILE = 16
_CH = 128

_N_PAD = 50176
_RPT = _N_PAD // _NTILE
_ZR = 392
_KSUB = 2
_SUP = _KSUB * _CH
_NSUP = 204
_E_PAD = _NTILE * _NSUP * _SUP
_EPT = _E_PAD // _NTILE

_BLK = 1024
_NBLK = _N_PAD // _BLK
_SROW = _N_PAD // 128

_MIN_I32 = np.int32(-2147483648)
_F32 = jnp.float32

_TRIU128 = np.triu(np.ones((128, 128), np.float32), 1)


def _sc_segment_sum(hz4, alive, combA, combB, z2d, z1d):
  mesh = plsc.VectorSubcoreMesh(
      core_axis_name="c", subcore_axis_name="s",
      num_cores=_NSC, num_subcores=_NTILE)
  out_type = (
      jax.ShapeDtypeStruct((_N_PAD, 128), _F32),
      jax.ShapeDtypeStruct((_N_PAD,), _F32),
      jax.ShapeDtypeStruct((_N_PAD,), _F32),
  )
  scratch = (
      [pltpu.VMEM((2 * _KSUB, _CH), jnp.int32) for _ in range(4)]
      + [pltpu.VMEM((_KSUB, _CH, _HF), _F32) for _ in range(3)]
      + [pltpu.VMEM((_KSUB, _CH), _F32) for _ in range(3)]
      + [pltpu.VMEM_SHARED((_N_PAD, _HF), _F32),
         pltpu.VMEM_SHARED((_N_PAD,), _F32)]
      + [pltpu.SemaphoreType.DMA for _ in range(16)]
  )

  @functools.partial(
      pl.kernel, out_type=out_type, mesh=mesh, scratch_types=scratch,
      compiler_params=pltpu.CompilerParams(use_tc_tiling_on_sc=False))
  def k(hz4_h, alive_h, combA_h, combB_h, z2d_h, z1d_h,
        ssum_h, degA_h, degB_h, *refs):
    ib = refs[0:4]
    valb = refs[4:7]
    ab = refs[7:10]
    acc, deg_acc = refs[10:12]
    isem = refs[12:16]
    gsem = refs[16:19]
    ssem = refs[19:22]
    asem = refs[22:25]
    dsem = refs[25:28]
    c = lax.axis_index("c")
    s = lax.axis_index("s")

    row0 = s * _RPT
    pltpu.sync_copy(z2d_h.at[pl.ds(row0, _RPT)], acc.at[pl.ds(row0, _RPT)])
    pltpu.sync_copy(z1d_h.at[pl.ds(row0, _RPT)],
                    deg_acc.at[pl.ds(row0, _RPT)])
    plsc.subcore_barrier()

    grp0 = s * _NSUP

    def idx_load(S, s4):
      rb = (grp0 + S) * 2 * _KSUB

      @pl.when(c == 0)
      def _():
        pltpu.async_copy(combA_h.at[pl.ds(rb, 2 * _KSUB)], ib[s4], isem[s4])

      @pl.when(c == 1)
      def _():
        pltpu.async_copy(combB_h.at[pl.ds(rb, 2 * _KSUB)], ib[s4], isem[s4])

    def gath(S, s4, s3):
      rb = (grp0 + S) * 2 * _KSUB
      pltpu.make_async_copy(combA_h.at[pl.ds(rb, 2 * _KSUB)],
                            ib[s4], isem[s4]).wait()
      for kk in range(_KSUB):
        pltpu.async_copy(hz4_h.at[ib[s4].at[kk]],
                         valb[s3].at[kk], gsem[s3])

      @pl.when(S % 2 == c)
      def _():
        for kk in range(_KSUB):
          pltpu.async_copy(alive_h.at[ib[s4].at[kk]],
                           ab[s3].at[kk], asem[s3])

    def process(S, s4, s3):
      for kk in range(_KSUB):
        pltpu.make_async_copy(hz4_h.at[ib[s4].at[kk]],
                              valb[s3].at[kk], gsem[s3]).wait()
        pltpu.async_copy(valb[s3].at[kk], acc.at[ib[s4].at[_KSUB + kk]],
                         ssem[s3], add=True)

      @pl.when(S % 2 == c)
      def _():
        for kk in range(_KSUB):
          pltpu.make_async_copy(alive_h.at[ib[s4].at[kk]],
                                ab[s3].at[kk], asem[s3]).wait()
          pltpu.async_copy(ab[s3].at[kk],
                           deg_acc.at[ib[s4].at[_KSUB + kk]],
                           dsem[s3], add=True)

    def drain_scat(S, s4, s3):
      for kk in range(_KSUB):
        pltpu.make_async_copy(valb[s3].at[kk],
                              acc.at[ib[s4].at[_KSUB + kk]],
                              ssem[s3]).wait()

      @pl.when(S % 2 == c)
      def _():
        for kk in range(_KSUB):
          pltpu.make_async_copy(ab[s3].at[kk],
                                deg_acc.at[ib[s4].at[_KSUB + kk]],
                                dsem[s3]).wait()

    idx_load(0, 0)
    idx_load(1, 1)
    idx_load(2, 2)
    gath(0, 0, 0)
    gath(1, 1, 1)

    def body(m, carry):
      for i in range(12):
        S = 12 * m + i
        process(S, i % 4, i % 3)

        @pl.when(S >= 1)
        def _():
          drain_scat(S - 1, (i + 3) % 4, (i + 2) % 3)

        @pl.when(S + 3 < _NSUP)
        def _():
          idx_load(S + 3, (i + 3) % 4)

        @pl.when(S + 2 < _NSUP)
        def _():
          gath(S + 2, (i + 2) % 4, (i + 2) % 3)
      return carry
    lax.fori_loop(0, _NSUP // 12, body, 0)
    drain_scat(_NSUP - 1, (_NSUP - 1) % 4, (_NSUP - 1) % 3)
    plsc.subcore_barrier()

    @pl.when(c == 0)
    def _():
      pltpu.sync_copy(acc.at[pl.ds(row0, _RPT)],
                      ssum_h.at[pl.ds(row0, _RPT), pl.ds(0, _HF)])
      pltpu.sync_copy(deg_acc.at[pl.ds(row0, _RPT)],
                      degA_h.at[pl.ds(row0, _RPT)])

    @pl.when(c == 1)
    def _():
      pltpu.sync_copy(acc.at[pl.ds(row0, _RPT)],
                      ssum_h.at[pl.ds(row0, _RPT), pl.ds(_HF, _HF)])
      pltpu.sync_copy(deg_acc.at[pl.ds(row0, _RPT)],
                      degB_h.at[pl.ds(row0, _RPT)])

  return k(hz4, alive, combA, combB, z2d, z1d)


def _gelu(t):
  return 0.5 * t * (1.0 + lax.erf(t * np.float32(1.0 / math.sqrt(2.0))))


def _tc_sage(ssum, degA, degB, hz128, alive, Wl, bl, Wr, p):
  def body(ss, dA, dB, hzb, al, wl, blv, wr, pv, hp_ref, sc_ref):
    d = jnp.maximum(dA[...] + dB[...], 1.0)
    inv = jnp.transpose((1.0 / d).reshape(1, _BLK))
    mean = ss[...][:, :_F] * inv
    hz = hzb[...][:, :_F]
    pre = (jnp.dot(mean, wl[...], preferred_element_type=_F32) + blv[...]
           + jnp.dot(hz, wr[...], preferred_element_type=_F32))
    hp = _gelu(pre)
    pn = pv[...]
    pn = pn * lax.rsqrt(jnp.sum(pn * pn))
    sco = jnp.tanh(jnp.sum(hp * pn, axis=1, keepdims=True))
    hp_ref[...] = hp
    sc_row = jnp.transpose(sco).reshape(_BLK // 128, 128)
    sc_ref[...] = jnp.where(al[...] > 0.0, sc_row, -2.0)

  nblock = lambda cols: pl.BlockSpec((_BLK, cols), lambda i: (i, 0))
  cblock = pl.BlockSpec((_BLK // 128, 128), lambda i: (i, 0))
  wblock = lambda r, c: pl.BlockSpec((r, c), lambda i: (0, 0))
  return pl.pallas_call(
      body,
      grid=(_NBLK,),
      in_specs=[nblock(128), cblock, cblock, nblock(128), cblock,
                wblock(_F, _F), wblock(1, _F), wblock(_F, _F), wblock(1, _F)],
      out_specs=[nblock(_F), cblock],
      out_shape=[jax.ShapeDtypeStruct((_N_PAD, _F), _F32),
                 jax.ShapeDtypeStruct((_SROW, 128), _F32)],
  )(ssum, degA, degB, hz128, alive, Wl, bl, Wr, p)


def _monotone_i32(bits):
  return jnp.where(bits >= 0, bits, bits ^ np.int32(0x7FFFFFFF))


def _tc_thresh(scg, kk):
  def body(sc_ref, thr_ref, tn_ref):
    mi = _monotone_i32(lax.bitcast_convert_type(sc_ref[...], jnp.int32))

    def step(i, P):
      bv = jnp.left_shift(np.int32(1), 31 - i)
      cand = P | bv
      cand_s = cand ^ _MIN_I32
      cnt = jnp.sum((mi >= cand_s).astype(jnp.int32))
      return jnp.where(cnt >= kk, cand, P)

    P = lax.fori_loop(0, 32, step, jnp.int32(0))
    T = P ^ _MIN_I32
    cg = jnp.sum((mi > T).astype(jnp.int32))
    thr_ref[...] = jnp.broadcast_to(T, (1, 1))
    tn_ref[...] = jnp.broadcast_to(kk - cg, (1, 1))

  return pl.pallas_call(
      body,
      out_shape=[jax.ShapeDtypeStruct((1, 1), jnp.int32),
                 jax.ShapeDtypeStruct((1, 1), jnp.int32)],
  )(scg)


def _tc_select(hp, sc, thr, tn, triu, kk):
  nrow = _BLK // 128

  def body(hp_ref, sc_ref, thr_ref, tn_ref, u_ref,
           hz_ref, al_ref, x_ref, cnt_ref):
    i = pl.program_id(0)

    @pl.when(i == 0)
    def _():
      cnt_ref[0] = np.int32(0)
      x_ref[...] = jnp.concatenate(
          [jnp.full((1, _F), -1e30, _F32), jnp.zeros((1, _F), _F32)], axis=1)

    scb = sc_ref[...]
    mi = _monotone_i32(lax.bitcast_convert_type(scb, jnp.int32))
    T = thr_ref[...]
    tnf = tn_ref[...].astype(_F32)
    eq = mi == T
    eqf = eq.astype(_F32)
    rows = jnp.sum(eqf, axis=1, keepdims=True)
    ri = lax.broadcasted_iota(jnp.int32, (nrow, nrow), 0)
    ci = lax.broadcasted_iota(jnp.int32, (nrow, nrow), 1)
    ls = (ri > ci).astype(_F32)
    rowpref = jnp.dot(ls, rows, preferred_element_type=_F32)
    inrow = jnp.dot(eqf, u_ref[...], preferred_element_type=_F32)
    base = cnt_ref[0].astype(_F32)
    pref = rowpref + inrow + base
    sel = (mi > T) | (eq & (pref < tnf))
    cnt_ref[0] = cnt_ref[0] + jnp.sum(eqf).astype(jnp.int32)

    selF = sel.astype(_F32)
    fac = jnp.transpose((selF * scb).reshape(1, _BLK))
    s512 = jnp.transpose(selF.reshape(1, _BLK))
    hz = hp_ref[...] * fac
    hz_ref[...] = jnp.concatenate(
        [hz, jnp.zeros((_BLK, 128 - _F), _F32)], axis=1)
    al_ref[...] = selF
    pmax = jnp.max(jnp.where(s512 > 0.0, hz, -1e30), axis=0, keepdims=True)
    psum = jnp.sum(hz, axis=0, keepdims=True)
    old = x_ref[...]
    x_ref[...] = jnp.concatenate(
        [jnp.maximum(old[:, :_F], pmax), old[:, _F:] + psum], axis=1)

    @pl.when(i == _NBLK - 1)
    def _():
      fin = x_ref[...]
      x_ref[...] = jnp.concatenate(
          [fin[:, :_F], fin[:, _F:] * np.float32(1.0 / kk)], axis=1)

  nblock = lambda cols: pl.BlockSpec((_BLK, cols), lambda i: (i, 0))
  cblock = pl.BlockSpec((nrow, 128), lambda i: (i, 0))
  full = lambda r, c: pl.BlockSpec((r, c), lambda i: (0, 0))
  return pl.pallas_call(
      body,
      grid=(_NBLK,),
      in_specs=[nblock(_F), cblock, full(1, 1), full(1, 1),
                full(128, 128)],
      out_specs=[nblock(128), cblock, full(1, 2 * _F)],
      out_shape=[jax.ShapeDtypeStruct((_N_PAD, 128), _F32),
                 jax.ShapeDtypeStruct((_SROW, 128), _F32),
                 jax.ShapeDtypeStruct((1, 2 * _F), _F32)],
      scratch_shapes=[pltpu.SMEM((1,), jnp.int32)],
  )(hp, sc, thr, tn, triu)


def _tc_mlp(z, W1, b1, W2, b2, W3, b3):
  def body(z_ref, w1, c1, w2, c2, w3, c3, out_ref):
    a = _gelu(jnp.dot(z_ref[...], w1[...], preferred_element_type=_F32)
              + c1[...])
    a = _gelu(jnp.dot(a, w2[...], preferred_element_type=_F32) + c2[...])
    out_ref[...] = jnp.dot(a, w3[...], preferred_element_type=_F32) + c3[...]

  return pl.pallas_call(
      body,
      out_shape=jax.ShapeDtypeStruct((1, 10), _F32),
  )(z, W1, b1, W2, b2, W3, b3)


def kernel(x, edge_index, batch, edge_attr, fields,
           Wl1, bl1, Wr1, Wl2, bl2, Wr2, Wl3, bl3, Wr3,
           p1, p2, p3, W1, b1, W2, b2, W3, b3):
  del batch, edge_attr
  npad = _N_PAD - _N
  h0 = jnp.concatenate([x[:, :3], fields], axis=1)
  hz128 = jnp.pad(h0, ((0, npad), (0, 128 - _F)))
  alive = jnp.pad(jnp.ones((_N,), _F32), (0, npad))

  epad = _E_PAD - _E
  extra = _N + (jnp.arange(epad, dtype=jnp.int32) % npad)
  src = jnp.concatenate([edge_index[0].astype(jnp.int32), extra])
  dst = jnp.concatenate([edge_index[1].astype(jnp.int32), extra])
  s3 = src.reshape(_NTILE * _NSUP, _KSUB, _CH)
  d3 = dst.reshape(_NTILE * _NSUP, _KSUB, _CH)
  combA = jnp.concatenate([4 * s3, d3], axis=1).reshape(-1, _CH)
  combB = jnp.concatenate([4 * s3 + 1, d3], axis=1).reshape(-1, _CH)

  triu = jnp.asarray(_TRIU128)
  z2d = jnp.zeros((_N_PAD, _HF), _F32)
  z1d = jnp.zeros((_N_PAD,), _F32)
  params = ((Wl1, bl1, Wr1, p1), (Wl2, bl2, Wr2, p2), (Wl3, bl3, Wr3, p3))
  nn = _N
  xs = []
  for lvl in range(3):
    kk = int(math.ceil(0.8 * nn))
    Wl, bl, Wr, p = params[lvl]
    hz4 = hz128.reshape(4 * _N_PAD, _HF)
    alive4 = jnp.repeat(alive, 4)
    ssum, degA, degB = _sc_segment_sum(hz4, alive4, combA, combB, z2d, z1d)
    hp, sc = _tc_sage(ssum,
                      degA.reshape(_SROW, 128), degB.reshape(_SROW, 128),
                      hz128, alive.reshape(_SROW, 128),
                      Wl, bl.reshape(1, _F), Wr, p.reshape(1, _F))
    thr, tn = _tc_thresh(sc, kk)
    hz128, alive1, xl = _tc_select(hp, sc, thr, tn, triu, kk)
    alive = alive1.reshape(_N_PAD)
    xs.append(xl)
    nn = kk

  z = xs[0] + xs[1] + xs[2]
  return _tc_mlp(z, W1, b1.reshape(1, _F), W2, b2.reshape(1, _F),
                 W3, b3.reshape(1, 10))
